# Initial kernel scaffold; baseline (speedup 1.0000x reference)
import jax
import jax.numpy as jnp
from jax.experimental import pallas as pl


def kernel(pos, rc_pair, elm_atoms, elm_table):
    raise NotImplementedError("write your pallas kernel here")


# speedup vs baseline: 1101.1125x; 1101.1125x over previous
import functools

import jax
import jax.numpy as jnp
from jax import lax
from jax.experimental import pallas as pl
from jax.experimental.pallas import tpu as pltpu
from jax.experimental.pallas import tpu_sc as plsc

RC = 6.0
L = 16

W_COEF = (
    1.0000000e+00, -2.4674010e+00, 2.0293560e+00, -6.6763014e-01,
    1.1766041e-01, -1.2892905e-02, 9.5216447e-04, -4.4345466e-05,
)


def _poly_w(v):
    w = jnp.full((L,), W_COEF[-1], dtype=jnp.float32)
    for c in W_COEF[-2::-1]:
        w = w * v + jnp.float32(c)
    return w


def _sc_coordination(posT, onehot, elm, tab, *, B, N, NUMEL, n_workers):
    wpb = n_workers // B
    rows = N // wpb
    n_chunks = N // L
    rc2 = jnp.float32(RC * RC)
    ncode = NUMEL * NUMEL

    mesh = plsc.VectorSubcoreMesh(core_axis_name="c", subcore_axis_name="s")
    info = plsc.get_sparse_core_info()
    nc = info.num_cores

    @functools.partial(
        pl.kernel,
        out_type=jax.ShapeDtypeStruct((n_workers, ncode * L), jnp.float32),
        mesh=mesh,
        compiler_params=pltpu.CompilerParams(needs_layout_passes=False),
        scratch_types=[
            pltpu.VMEM((3 * N,), jnp.float32),
            pltpu.VMEM((NUMEL * N,), jnp.float32),
            pltpu.VMEM((rows,), jnp.int32),
            pltpu.VMEM((2 * L,), jnp.float32),
            pltpu.VMEM((ncode * L,), jnp.float32),
        ],
    )
    def k(posT_hbm, oh_hbm, elm_hbm, tab_hbm, out_hbm, posv, ohv, elmv, tabv, outv):
        wid = lax.axis_index("s") * nc + lax.axis_index("c")
        b = wid // wpb
        r0 = (wid % wpb) * rows
        pltpu.sync_copy(posT_hbm.at[b], posv)
        pltpu.sync_copy(oh_hbm.at[b], ohv)
        pltpu.sync_copy(elm_hbm.at[b, pl.ds(r0, rows)], elmv)
        pltpu.sync_copy(tab_hbm, tabv)

        e0 = jnp.where(lax.iota(jnp.int32, L) == 0,
                       jnp.float32(1.0), jnp.float32(0.0))
        zero = jnp.zeros((L,), jnp.float32)
        zi16 = jnp.zeros((L,), jnp.int32)

        def row_body(ii, totals):
            iv = jnp.full((L,), r0 + ii, jnp.int32)
            xi = plsc.load_gather(posv, [iv])
            yi = plsc.load_gather(posv, [iv + N])
            zi = plsc.load_gather(posv, [iv + 2 * N])
            eib = plsc.load_gather(elmv, [jnp.full((L,), ii, jnp.int32)])
            c0 = plsc.load_gather(tabv, [eib * 3])
            c1 = plsc.load_gather(tabv, [eib * 3 + 1])
            c2 = plsc.load_gather(tabv, [eib * 3 + 2])

            def chunk_body(jc, accs):
                a0, a1, a2 = accs
                j = jc * L
                xj = posv[pl.ds(j, L)]
                yj = posv[pl.ds(j + N, L)]
                zj = posv[pl.ds(j + 2 * N, L)]
                dx = xi - xj
                dy = yi - yj
                dz = zi - zj
                sod = dx * dx + dy * dy + dz * dz
                o0 = ohv[pl.ds(j, L)]
                o1 = ohv[pl.ds(j + N, L)]
                o2 = ohv[pl.ds(j + 2 * N, L)]
                inv = c0 * o0 + c1 * o1 + c2 * o2
                v = jnp.minimum(sod * inv, jnp.float32(1.0))
                w = _poly_w(v)
                wm = jnp.where(sod < rc2, w, jnp.float32(0.0))
                return (a0 + wm * o0, a1 + wm * o1, a2 + wm * o2)

            a0, a1, a2 = lax.fori_loop(
                0, n_chunks, chunk_body, (zero, zero, zero))

            vlds = [plsc.load_gather(tabv, [eib * 3 + c + L])
                    for c in range(3)]
            deltas = []
            for c, acc_c in enumerate((a0, a1, a2)):
                selfw = (eib == c).astype(jnp.float32)
                deltas.append((acc_c - selfw * e0) * vlds[c])
            new_totals = []
            for k9 in range(ncode):
                er, c = divmod(k9, 3)
                m = (eib == er).astype(jnp.float32)
                new_totals.append(totals[k9] + m * deltas[c])
            return tuple(new_totals)

        totals = lax.fori_loop(
            0, rows, row_body, tuple(zero for _ in range(ncode)))
        for k9 in range(ncode):
            outv[pl.ds(k9 * L, L)] = totals[k9]
        pltpu.sync_copy(outv, out_hbm.at[wid])

    return k(posT, onehot, elm, tab)


def kernel(pos, rc_pair, elm_atoms, elm_table):
    B, N, _ = pos.shape
    n_types = rc_pair.shape[0]
    NUMEL = elm_table.shape[0]
    n_workers = 32

    posT = pos.astype(jnp.float32).transpose(0, 2, 1).reshape(B, 3 * N)
    ea = elm_atoms.astype(jnp.int32)
    onehot = (ea[:, None, :] == jnp.arange(NUMEL, dtype=jnp.int32)[None, :, None]
              ).astype(jnp.float32).reshape(B, NUMEL * N)
    etf = elm_table.reshape(-1).astype(jnp.int32)
    validf = (etf >= 0).astype(jnp.float32)
    rcp = jnp.where(etf >= 0, rc_pair[jnp.maximum(etf, 0)], jnp.float32(1.0))
    inv2 = 1.0 / (rcp * rcp)
    pad = 16 - etf.shape[0]
    tab = jnp.concatenate([jnp.pad(inv2, (0, pad)), jnp.pad(validf, (0, pad))])

    parts = _sc_coordination(posT, onehot, ea, tab,
                             B=B, N=N, NUMEL=NUMEL, n_workers=n_workers)
    per_code = parts.reshape(B, n_workers // B, NUMEL * NUMEL, 16).sum((1, 3))
    code2type = (etf[:, None] == jnp.arange(n_types, dtype=jnp.int32)[None, :]
                 ).astype(jnp.float32)
    return (per_code @ code2type) * jnp.float32(0.5)
